# Initial kernel scaffold; baseline (speedup 1.0000x reference)
import jax
import jax.numpy as jnp
from jax.experimental import pallas as pl


def kernel(x_in, edge_index_in, W1, b1, W2, b2, W3, b3, W4, b4, g1, be1, g5, be5):
    raise NotImplementedError("write your pallas kernel here")


# speedup vs baseline: 4.0985x; 4.0985x over previous
import functools

import jax
import jax.numpy as jnp
from jax import lax
from jax.experimental import pallas as pl
from jax.experimental.pallas import tpu as pltpu
from jax.experimental.pallas import tpu_sc as plsc

N = 10000
E = 320000
D = 128
NC = 2
NS = 16
NW = NC * NS
CHUNK = 128
CH = ((E + NW - 1) // NW + CHUNK - 1) // CHUNK
EPT = CH * CHUNK
E_PAD = NW * EPT
AR = 10240
RPT = AR // NS

@functools.cache
def _sc_aggregate_fn():
    mesh = plsc.VectorSubcoreMesh(core_axis_name="c", subcore_axis_name="s")

    @functools.partial(
        pl.kernel,
        mesh=mesh,
        out_type=jax.ShapeDtypeStruct((NC, AR, D), jnp.float32),
        scratch_types=[
            pltpu.VMEM((CH, CHUNK), jnp.int32),
            pltpu.VMEM((CH, CHUNK), jnp.int32),
            pltpu.VMEM((CHUNK, D), jnp.float32),
            pltpu.VMEM_SHARED((AR, D), jnp.float32),
            pltpu.SemaphoreType.DMA,
        ],
    )
    def _sc_aggregate(table_hbm, src_hbm, dst_hbm, zeros_hbm, out_hbm,
                      src_idx, dst_idx, rows, acc, sem):
        c = lax.axis_index("c")
        s = lax.axis_index("s")
        wid = c * NS + s
        base = s * RPT

        @pl.when(c == 0)
        def _():
            pltpu.sync_copy(table_hbm.at[pl.ds(base, RPT)],
                            acc.at[pl.ds(base, RPT)])

        @pl.when(c == 1)
        def _():
            pltpu.sync_copy(zeros_hbm, acc.at[pl.ds(base, RPT)])

        pltpu.sync_copy(src_hbm.at[wid], src_idx)
        pltpu.sync_copy(dst_hbm.at[wid], dst_idx)
        plsc.subcore_barrier()

        def body(j, carry):
            pltpu.async_copy(table_hbm.at[src_idx.at[j]], rows, sem).wait()
            pltpu.sync_copy(rows, acc.at[dst_idx.at[j]], add=True)
            return carry

        lax.fori_loop(0, CH, body, 0)
        plsc.subcore_barrier()

        pltpu.sync_copy(acc.at[pl.ds(base, RPT)],
                        out_hbm.at[c, pl.ds(base, RPT)])

    return _sc_aggregate


def _tc_mlp_bn(p0_ref, p1_ref, wa_ref, ba_ref, wb_ref, bb_ref, g_ref, be_ref,
               out_ref):
    h = p0_ref[...][:N] + p1_ref[...][:N]
    h = jnp.maximum(jnp.dot(h, wa_ref[...],
                            preferred_element_type=jnp.float32) + ba_ref[...], 0.0)
    h = jnp.dot(h, wb_ref[...], preferred_element_type=jnp.float32) + bb_ref[...]
    h = jnp.maximum(h, 0.0)
    mean = jnp.mean(h, axis=0, keepdims=True)
    ctr = h - mean
    var = jnp.mean(ctr * ctr, axis=0, keepdims=True)
    out_ref[...] = ctr * lax.rsqrt(var + 1e-5) * g_ref[...] + be_ref[...]


_tc_call = pl.pallas_call(
    _tc_mlp_bn,
    out_shape=jax.ShapeDtypeStruct((N, D), jnp.float32),
)


def kernel(x_in, edge_index_in, W1, b1, W2, b2, W3, b3, W4, b4, g1, be1, g5, be5):
    src = edge_index_in[0]
    dst = edge_index_in[1]
    pad = E_PAD - E
    src3 = jnp.concatenate([src, jnp.full((pad,), N, jnp.int32)]).reshape(NW, CH, CHUNK)
    dst3 = jnp.concatenate([dst, jnp.zeros((pad,), jnp.int32)]).reshape(NW, CH, CHUNK)
    zeros = jnp.zeros((RPT, D), jnp.float32)

    sc_agg = _sc_aggregate_fn()
    x_pad = jnp.pad(x_in, ((0, AR - N), (0, 0)))
    parts1 = sc_agg(x_pad, src3, dst3, zeros)
    hid = _tc_call(parts1[0], parts1[1],
                   W1.T, b1[None, :], W2.T, b2[None, :],
                   g1[None, :], be1[None, :])

    hid_pad = jnp.pad(hid, ((0, AR - N), (0, 0)))
    parts2 = sc_agg(hid_pad, src3, dst3, zeros)
    W4p = jnp.pad(W4, ((0, D - W4.shape[0]), (0, 0)))
    b4p = jnp.pad(b4, (0, D - b4.shape[0]))
    g5p = jnp.pad(g5, (0, D - g5.shape[0]))
    be5p = jnp.pad(be5, (0, D - be5.shape[0]))
    h2 = _tc_call(parts2[0], parts2[1],
                  W3.T, b3[None, :], W4p.T, b4p[None, :],
                  g5p[None, :], be5p[None, :])
    return (h2[:, : W4.shape[0]], hid)
